# baseline (speedup 1.0000x reference)
import functools

import jax
import jax.numpy as jnp
from jax import lax
from jax.experimental import pallas as pl
from jax.experimental.pallas import tpu as pltpu
from jax.experimental.pallas import tpu_sc as plsc

NC = 2
NS = 16
NW = NC * NS
LANES = 16
CH = 12800


def _rowsum_kernel(x, n_pad, grid):
    def body(x_ref, rs_ref):
        rs_ref[...] = jnp.sum(x_ref[...], axis=-1)

    d = x.shape[1]
    blk_n = n_pad // grid
    nbv = -(-x.shape[0] // blk_n)
    return pl.pallas_call(
        body,
        grid=(grid,),
        in_specs=[
            pl.BlockSpec((blk_n, d), lambda i: (jnp.minimum(i, nbv - 1), 0)),
        ],
        out_specs=pl.BlockSpec((blk_n,), lambda i: (i,)),
        out_shape=jax.ShapeDtypeStruct((n_pad,), jnp.float32),
    )(x)


def _fused_sc_kernel(ei, rs_pad, *, e, n_pad, nch, seg):
    mesh = plsc.VectorSubcoreMesh(core_axis_name="c", subcore_axis_name="s")

    @functools.partial(
        pl.kernel,
        mesh=mesh,
        out_type=jax.ShapeDtypeStruct((e,), jnp.float32),
        scratch_types=[
            pltpu.VMEM((2, CH), jnp.int32),
            pltpu.VMEM((CH,), jnp.int32),
            pltpu.VMEM((CH,), jnp.float32),
            pltpu.VMEM((seg,), jnp.float32),
            pltpu.VMEM((seg,), jnp.float32),
            pltpu.VMEM((n_pad,), jnp.float32),
            pltpu.VMEM((n_pad,), jnp.float32),
            pltpu.VMEM((2, CH), jnp.int32),
            pltpu.VMEM((CH,), jnp.float32),
            pltpu.VMEM_SHARED((n_pad,), jnp.float32),
            pltpu.VMEM_SHARED((n_pad,), jnp.float32),
            pltpu.VMEM_SHARED((n_pad,), jnp.float32),
            pltpu.SemaphoreType.DMA,
            pltpu.SemaphoreType.DMA,
            pltpu.SemaphoreType.DMA,
        ],
        compiler_params=pltpu.CompilerParams(needs_layout_passes=False),
    )
    def k(ei_hbm, rs_hbm, out_hbm, h2_v, to1_v, ones_v, seg_v, rs_v, s_v,
          tt_v, g2_v, o_v, deg_sh, s_sh, t_sh, sem_h, sem_g, sem_r):
        cid = lax.axis_index("c")
        sid = lax.axis_index("s")
        wid = cid * NS + sid
        sbase = pl.multiple_of(sid * seg, 8)
        hbase = pl.multiple_of(sid * CH, 128)
        gbase = pl.multiple_of(wid * CH, 128)

        hist_cp = pltpu.async_copy(ei_hbm.at[:, pl.ds(hbase, CH)], h2_v,
                                   sem_h)
        rs_cp = pltpu.async_copy(rs_hbm.at[pl.ds(sbase, seg)], rs_v, sem_r)

        @pl.when(wid < nch)
        def _():
            pltpu.async_copy(ei_hbm.at[:, pl.ds(gbase, CH)], g2_v, sem_g)

        ones16 = jnp.full((LANES,), 1.0, jnp.float32)
        zeros16 = jnp.zeros((LANES,), jnp.float32)

        @plsc.parallel_loop(0, seg // LANES, unroll=8)
        def _(i):
            seg_v[pl.ds(i * LANES, LANES)] = zeros16

        pltpu.sync_copy(seg_v, deg_sh.at[pl.ds(sbase, seg)])

        @plsc.parallel_loop(0, CH // LANES, unroll=8)
        def _(i):
            ones_v[pl.ds(i * LANES, LANES)] = ones16

        hist_cp.wait()

        @plsc.parallel_loop(0, CH // LANES, unroll=8)
        def _(i):
            sl = pl.ds(i * LANES, LANES)
            to1_v[sl] = h2_v[1, sl]

        plsc.subcore_barrier()

        pltpu.sync_copy(ones_v, deg_sh.at[to1_v], add=True)

        @pl.when(sid + NS < nch)
        def _():
            h2 = pl.multiple_of((sid + NS) * CH, 128)
            pltpu.sync_copy(ei_hbm.at[:, pl.ds(h2, CH)], h2_v)

            @plsc.parallel_loop(0, CH // LANES, unroll=8)
            def _(i):
                sl = pl.ds(i * LANES, LANES)
                to1_v[sl] = h2_v[1, sl]

            pltpu.sync_copy(ones_v, deg_sh.at[to1_v], add=True)

        plsc.subcore_barrier()

        pltpu.sync_copy(deg_sh.at[pl.ds(sbase, seg)], seg_v)
        rs_cp.wait()

        @plsc.parallel_loop(0, seg // LANES, unroll=4)
        def _(i):
            sl = pl.ds(i * LANES, LANES)
            d = seg_v[sl]
            ibits = plsc.bitcast(d, jnp.int32)
            y = plsc.bitcast(
                jnp.full((LANES,), 0x5F3759DF, jnp.int32)
                - lax.shift_right_logical(ibits, 1),
                jnp.float32,
            )
            hd = 0.5 * d
            y = y * (1.5 - hd * y * y)
            y = y * (1.5 - hd * y * y)
            y = y * (1.5 - hd * y * y)
            t = jnp.where(d == 0.0, 0.0, y)
            seg_v[sl] = t
            rs_v[sl] = t * rs_v[sl]

        pltpu.sync_copy(rs_v, s_sh.at[pl.ds(sbase, seg)])
        pltpu.sync_copy(seg_v, t_sh.at[pl.ds(sbase, seg)])
        plsc.subcore_barrier()

        @pl.when(wid < nch)
        def _():
            pltpu.sync_copy(s_sh, s_v)
            pltpu.sync_copy(t_sh, tt_v)
            pltpu.make_async_copy(ei_hbm.at[:, pl.ds(gbase, CH)], g2_v,
                                  sem_g).wait()

            @plsc.parallel_loop(0, CH // LANES, unroll=8)
            def _(i):
                sl = pl.ds(i * LANES, LANES)
                sv = plsc.load_gather(s_v, [g2_v[0, sl]])
                tv = plsc.load_gather(tt_v, [g2_v[1, sl]])
                o_v[sl] = sv * tv

            obase = pl.multiple_of(wid * CH, 8)
            pltpu.sync_copy(o_v, out_hbm.at[pl.ds(obase, CH)])

    return k(ei, rs_pad)


def kernel(x, edge_index):
    n, d = x.shape
    e = edge_index.shape[1]
    assert e % CH == 0, "edge count must split into chunks"
    nch = e // CH
    assert NS <= nch <= NW
    n_pad = -(-n // (NS * LANES)) * (NS * LANES)
    while n_pad % nch != 0:
        n_pad += NS * LANES
    seg = n_pad // NS

    rs_pad = _rowsum_kernel(x, n_pad, nch)
    return _fused_sc_kernel(edge_index.astype(jnp.int32), rs_pad, e=e,
                            n_pad=n_pad, nch=nch, seg=seg)


# speedup vs baseline: 1.1929x; 1.1929x over previous
import functools

import jax
import jax.numpy as jnp
from jax import lax
from jax.experimental import pallas as pl
from jax.experimental.pallas import tpu as pltpu
from jax.experimental.pallas import tpu_sc as plsc

NC = 2
NS = 16
NW = NC * NS
LANES = 16
CH = 12800


def _rowsum_kernel(x, n_pad, grid):
    def body(x_ref, rs_ref):
        rs_ref[...] = jnp.sum(x_ref[...], axis=-1)

    d = x.shape[1]
    blk_n = n_pad // grid
    assert (grid - 1) * blk_n < x.shape[0]
    return pl.pallas_call(
        body,
        grid=(grid,),
        in_specs=[pl.BlockSpec((blk_n, d), lambda i: (i, 0))],
        out_specs=pl.BlockSpec((blk_n,), lambda i: (i,)),
        out_shape=jax.ShapeDtypeStruct((n_pad,), jnp.float32),
    )(x)


def _fused_sc_kernel(ei, rs_pad, *, e, n_pad, nch, seg):
    mesh = plsc.VectorSubcoreMesh(core_axis_name="c", subcore_axis_name="s")

    @functools.partial(
        pl.kernel,
        mesh=mesh,
        out_type=jax.ShapeDtypeStruct((e,), jnp.float32),
        scratch_types=[
            pltpu.VMEM((2, CH), jnp.int32),
            pltpu.VMEM((CH,), jnp.int32),
            pltpu.VMEM((CH,), jnp.float32),
            pltpu.VMEM((seg,), jnp.float32),
            pltpu.VMEM((seg,), jnp.float32),
            pltpu.VMEM((n_pad,), jnp.float32),
            pltpu.VMEM((n_pad,), jnp.float32),
            pltpu.VMEM((2, CH), jnp.int32),
            pltpu.VMEM((CH,), jnp.float32),
            pltpu.VMEM_SHARED((n_pad,), jnp.float32),
            pltpu.VMEM_SHARED((n_pad,), jnp.float32),
            pltpu.VMEM_SHARED((n_pad,), jnp.float32),
            pltpu.SemaphoreType.DMA,
            pltpu.SemaphoreType.DMA,
            pltpu.SemaphoreType.DMA,
        ],
        compiler_params=pltpu.CompilerParams(needs_layout_passes=False),
    )
    def k(ei_hbm, rs_hbm, out_hbm, h2_v, to1_v, ones_v, seg_v, rs_v, s_v,
          tt_v, g2_v, o_v, deg_sh, s_sh, t_sh, sem_h, sem_g, sem_r):
        cid = lax.axis_index("c")
        sid = lax.axis_index("s")
        wid = cid * NS + sid
        sbase = pl.multiple_of(sid * seg, 8)
        hbase = pl.multiple_of(sid * CH, 128)
        gbase = pl.multiple_of(wid * CH, 128)

        hist_cp = pltpu.async_copy(ei_hbm.at[:, pl.ds(hbase, CH)], h2_v,
                                   sem_h)
        rs_cp = pltpu.async_copy(rs_hbm.at[pl.ds(sbase, seg)], rs_v, sem_r)

        @pl.when(wid < nch)
        def _():
            pltpu.async_copy(ei_hbm.at[:, pl.ds(gbase, CH)], g2_v, sem_g)

        ones16 = jnp.full((LANES,), 1.0, jnp.float32)
        zeros16 = jnp.zeros((LANES,), jnp.float32)

        @plsc.parallel_loop(0, seg // LANES, unroll=8)
        def _(i):
            seg_v[pl.ds(i * LANES, LANES)] = zeros16

        pltpu.sync_copy(seg_v, deg_sh.at[pl.ds(sbase, seg)])

        @plsc.parallel_loop(0, CH // LANES, unroll=8)
        def _(i):
            ones_v[pl.ds(i * LANES, LANES)] = ones16

        hist_cp.wait()

        @plsc.parallel_loop(0, CH // LANES, unroll=8)
        def _(i):
            sl = pl.ds(i * LANES, LANES)
            to1_v[sl] = h2_v[1, sl]

        plsc.subcore_barrier()

        pltpu.sync_copy(ones_v, deg_sh.at[to1_v], add=True)

        @pl.when(sid + NS < nch)
        def _():
            h2 = pl.multiple_of((sid + NS) * CH, 128)
            pltpu.sync_copy(ei_hbm.at[:, pl.ds(h2, CH)], h2_v)

            @plsc.parallel_loop(0, CH // LANES, unroll=8)
            def _(i):
                sl = pl.ds(i * LANES, LANES)
                to1_v[sl] = h2_v[1, sl]

            pltpu.sync_copy(ones_v, deg_sh.at[to1_v], add=True)

        plsc.subcore_barrier()

        pltpu.sync_copy(deg_sh.at[pl.ds(sbase, seg)], seg_v)
        rs_cp.wait()

        @plsc.parallel_loop(0, seg // LANES, unroll=4)
        def _(i):
            sl = pl.ds(i * LANES, LANES)
            d = seg_v[sl]
            ibits = plsc.bitcast(d, jnp.int32)
            y = plsc.bitcast(
                jnp.full((LANES,), 0x5F3759DF, jnp.int32)
                - lax.shift_right_logical(ibits, 1),
                jnp.float32,
            )
            hd = 0.5 * d
            y = y * (1.5 - hd * y * y)
            y = y * (1.5 - hd * y * y)
            y = y * (1.5 - hd * y * y)
            t = jnp.where(d == 0.0, 0.0, y)
            seg_v[sl] = t
            rs_v[sl] = t * rs_v[sl]

        pltpu.sync_copy(rs_v, s_sh.at[pl.ds(sbase, seg)])
        pltpu.sync_copy(seg_v, t_sh.at[pl.ds(sbase, seg)])
        plsc.subcore_barrier()

        @pl.when(wid < nch)
        def _():
            pltpu.sync_copy(s_sh, s_v)
            pltpu.sync_copy(t_sh, tt_v)
            pltpu.make_async_copy(ei_hbm.at[:, pl.ds(gbase, CH)], g2_v,
                                  sem_g).wait()

            @plsc.parallel_loop(0, CH // LANES, unroll=8)
            def _(i):
                sl = pl.ds(i * LANES, LANES)
                sv = plsc.load_gather(s_v, [g2_v[0, sl]])
                tv = plsc.load_gather(tt_v, [g2_v[1, sl]])
                o_v[sl] = sv * tv

            obase = pl.multiple_of(wid * CH, 8)
            pltpu.sync_copy(o_v, out_hbm.at[pl.ds(obase, CH)])

    return k(ei, rs_pad)


def kernel(x, edge_index):
    n, d = x.shape
    e = edge_index.shape[1]
    assert e % CH == 0, "edge count must split into chunks"
    nch = e // CH
    assert NS <= nch <= NW
    n_pad = -(-n // (NS * LANES)) * (NS * LANES)
    while n_pad % nch != 0:
        n_pad += NS * LANES
    seg = n_pad // NS

    n_rs = -(-n // 2048) * 2048
    rs_pad = jnp.concatenate([
        _rowsum_kernel(x, n_rs, n_rs // 2048),
        jnp.zeros((n_pad - n_rs,), jnp.float32),
    ])
    return _fused_sc_kernel(edge_index.astype(jnp.int32), rs_pad, e=e,
                            n_pad=n_pad, nch=nch, seg=seg)


# speedup vs baseline: 1.2903x; 1.0817x over previous
import functools

import jax
import jax.numpy as jnp
from jax import lax
from jax.experimental import pallas as pl
from jax.experimental.pallas import tpu as pltpu
from jax.experimental.pallas import tpu_sc as plsc

NC = 2
NS = 16
NW = NC * NS
LANES = 16


def _rowsum_kernel(x, n_pad):
    def body(x_ref, out_ref):
        out_ref[...] = jnp.sum(x_ref[...], axis=-1)

    d = x.shape[1]
    blk = 2048
    assert n_pad % blk == 0 and n_pad - blk < x.shape[0], \
        "every block must overlap valid rows"
    return pl.pallas_call(
        body,
        grid=(n_pad // blk,),
        in_specs=[pl.BlockSpec((blk, d), lambda i: (i, 0))],
        out_specs=pl.BlockSpec((blk,), lambda i: (i,)),
        out_shape=jax.ShapeDtypeStruct((n_pad,), jnp.float32),
    )(x)


def _fused_sc_kernel(ei_flat, rs_pad, *, e, n_pad, cpw, seg):
    mesh = plsc.VectorSubcoreMesh(core_axis_name="c", subcore_axis_name="s")

    @functools.partial(
        pl.kernel,
        mesh=mesh,
        out_type=jax.ShapeDtypeStruct((e,), jnp.float32),
        scratch_types=[
            pltpu.VMEM((2 * cpw,), jnp.int32),
            pltpu.VMEM((2 * cpw,), jnp.float32),
            pltpu.VMEM((seg,), jnp.float32),
            pltpu.VMEM((seg,), jnp.float32),
            pltpu.VMEM((n_pad,), jnp.float32),
            pltpu.VMEM((n_pad,), jnp.float32),
            pltpu.VMEM((cpw,), jnp.int32),
            pltpu.VMEM((cpw,), jnp.int32),
            pltpu.VMEM((cpw,), jnp.float32),
            pltpu.VMEM_SHARED((n_pad,), jnp.float32),
            pltpu.VMEM_SHARED((n_pad,), jnp.float32),
            pltpu.VMEM_SHARED((n_pad,), jnp.float32),
            pltpu.SemaphoreType.DMA,
            pltpu.SemaphoreType.DMA,
            pltpu.SemaphoreType.DMA,
            pltpu.SemaphoreType.DMA,
        ],
        compiler_params=pltpu.CompilerParams(needs_layout_passes=False),
    )
    def k(ei_hbm, rs_hbm, out_hbm, idx_v, ones_v, seg_v, rs_v, s_v, tt_v,
          fi_v, ti_v, o_v, deg_sh, s_sh, t_sh, sem_h, sem_f, sem_t, sem_r):
        cid = lax.axis_index("c")
        sid = lax.axis_index("s")
        wid = cid * NS + sid
        sbase = pl.multiple_of(sid * seg, 8)
        ebase = pl.multiple_of(wid * cpw, 8)
        hbase = pl.multiple_of(e + sid * (2 * cpw), 8)

        hist_cp = pltpu.async_copy(ei_hbm.at[pl.ds(hbase, 2 * cpw)], idx_v,
                                   sem_h)
        from_cp = pltpu.async_copy(ei_hbm.at[pl.ds(ebase, cpw)], fi_v, sem_f)
        to_cp = pltpu.async_copy(ei_hbm.at[pl.ds(e + ebase, cpw)], ti_v,
                                 sem_t)
        rs_cp = pltpu.async_copy(rs_hbm.at[pl.ds(sbase, seg)], rs_v, sem_r)

        ones16 = jnp.full((LANES,), 1.0, jnp.float32)
        zeros16 = jnp.zeros((LANES,), jnp.float32)

        @plsc.parallel_loop(0, seg // LANES, unroll=8)
        def _(i):
            seg_v[pl.ds(i * LANES, LANES)] = zeros16

        pltpu.sync_copy(seg_v, deg_sh.at[pl.ds(sbase, seg)])

        @plsc.parallel_loop(0, (2 * cpw) // LANES, unroll=8)
        def _(i):
            ones_v[pl.ds(i * LANES, LANES)] = ones16

        hist_cp.wait()
        plsc.subcore_barrier()

        pltpu.sync_copy(ones_v, deg_sh.at[idx_v], add=True)
        plsc.subcore_barrier()

        pltpu.sync_copy(deg_sh.at[pl.ds(sbase, seg)], seg_v)
        rs_cp.wait()

        @plsc.parallel_loop(0, seg // LANES, unroll=4)
        def _(i):
            sl = pl.ds(i * LANES, LANES)
            d = seg_v[sl]
            ibits = plsc.bitcast(d, jnp.int32)
            y = plsc.bitcast(
                jnp.full((LANES,), 0x5F3759DF, jnp.int32)
                - lax.shift_right_logical(ibits, 1),
                jnp.float32,
            )
            hd = 0.5 * d
            y = y * (1.5 - hd * y * y)
            y = y * (1.5 - hd * y * y)
            y = y * (1.5 - hd * y * y)
            t = jnp.where(d == 0.0, 0.0, y)
            seg_v[sl] = t
            rs_v[sl] = t * rs_v[sl]

        pltpu.sync_copy(rs_v, s_sh.at[pl.ds(sbase, seg)])
        pltpu.sync_copy(seg_v, t_sh.at[pl.ds(sbase, seg)])
        plsc.subcore_barrier()

        pltpu.sync_copy(s_sh, s_v)
        pltpu.sync_copy(t_sh, tt_v)
        from_cp.wait()
        to_cp.wait()

        @plsc.parallel_loop(0, cpw // LANES, unroll=8)
        def _(i):
            sl = pl.ds(i * LANES, LANES)
            sv = plsc.load_gather(s_v, [fi_v[sl]])
            tv = plsc.load_gather(tt_v, [ti_v[sl]])
            o_v[sl] = sv * tv

        pltpu.sync_copy(o_v, out_hbm.at[pl.ds(ebase, cpw)])

    return k(ei_flat, rs_pad)


def kernel(x, edge_index):
    n, d = x.shape
    e = edge_index.shape[1]
    assert e % (NW * LANES) == 0, "edge count must split across subcores"
    cpw = e // NW
    n_pad = -(-n // (NS * LANES)) * (NS * LANES)
    seg = n_pad // NS

    ei_flat = edge_index.astype(jnp.int32).reshape(2 * e)
    rs_pad = _rowsum_kernel(x, n_pad)
    return _fused_sc_kernel(ei_flat, rs_pad, e=e, n_pad=n_pad, cpw=cpw, seg=seg)
